# baseline (speedup 1.0000x reference)
import jax
import jax.numpy as jnp
from jax.experimental import pallas as pl
from jax.experimental.pallas import tpu as pltpu

B = 16384
D = 64
H = 32
G = 18
BT = 1024


def _fused_kernel(x_ref, gv_ref, w1_ref, b1_ref, w2bd_ref, b2_ref,
                  w3r_ref, b3_ref, wa_ref, ba_ref, e_ref,
                  wgx_ref, wgr_ref, bg_ref, out_ref):
    x = x_ref[:]
    gv = gv_ref[:]

    logits = jnp.dot(x, wa_ref[:], preferred_element_type=jnp.float32) + ba_ref[:]
    m = jnp.max(logits, axis=-1, keepdims=True)
    e = jnp.exp(logits - m)
    gw = e / jnp.sum(e, axis=-1, keepdims=True)
    c = gw * gv

    h1 = jnp.maximum(
        jnp.dot(x, w1_ref[:], preferred_element_type=jnp.float32) + b1_ref[:],
        0.0)
    h2 = jnp.maximum(
        jnp.dot(h1, w2bd_ref[:], preferred_element_type=jnp.float32) + b2_ref[:],
        0.0)
    cexp = jnp.dot(c, e_ref[:], preferred_element_type=jnp.float32)
    u = h2 * cexp
    r = (jnp.dot(u, w3r_ref[:], preferred_element_type=jnp.float32)
         + jnp.dot(c, b3_ref[:], preferred_element_type=jnp.float32))

    out = jnp.maximum(
        jnp.dot(x, wgx_ref[:], preferred_element_type=jnp.float32)
        + jnp.dot(r, wgr_ref[:], preferred_element_type=jnp.float32)
        + bg_ref[:],
        0.0)
    out_ref[:] = out


def kernel(item_features, genre_vectors, W1, b1, W2, b2, W3, b3, Wa, ba, Wg, bg):
    w1r = jnp.transpose(W1, (1, 0, 2)).reshape(D, G * 2 * H)
    b1r = b1.reshape(1, G * 2 * H)
    eyeg = jnp.eye(G, dtype=jnp.float32)
    w2bd = (W2[:, :, None, :] * eyeg[:, None, :, None]).reshape(G * 2 * H, G * H)
    b2r = b2.reshape(1, G * H)
    w3r = W3.reshape(G * H, H)
    e_expand = jnp.repeat(eyeg, H, axis=1)
    wgx = Wg[:D]
    wgr = Wg[D:]
    bar = ba.reshape(1, G)
    bgr = bg.reshape(1, D)

    grid = (B // BT,)
    full = lambda i: (0, 0)
    out = pl.pallas_call(
        _fused_kernel,
        grid=grid,
        in_specs=[
            pl.BlockSpec((BT, D), lambda i: (i, 0)),
            pl.BlockSpec((BT, G), lambda i: (i, 0)),
            pl.BlockSpec((D, G * 2 * H), full),
            pl.BlockSpec((1, G * 2 * H), full),
            pl.BlockSpec((G * 2 * H, G * H), full),
            pl.BlockSpec((1, G * H), full),
            pl.BlockSpec((G * H, H), full),
            pl.BlockSpec((G, H), full),
            pl.BlockSpec((D, G), full),
            pl.BlockSpec((1, G), full),
            pl.BlockSpec((G, G * H), full),
            pl.BlockSpec((D, D), full),
            pl.BlockSpec((H, D), full),
            pl.BlockSpec((1, D), full),
        ],
        out_specs=pl.BlockSpec((BT, D), lambda i: (i, 0)),
        out_shape=jax.ShapeDtypeStruct((B, D), jnp.float32),
        compiler_params=pltpu.CompilerParams(
            dimension_semantics=("arbitrary",),
        ),
    )(item_features, genre_vectors, w1r, b1r, w2bd, b2r, w3r, b3,
      Wa, bar, e_expand, wgx, wgr, bgr)
    return out


# speedup vs baseline: 1.3158x; 1.3158x over previous
import jax
import jax.numpy as jnp
from jax.experimental import pallas as pl
from jax.experimental.pallas import tpu as pltpu

B = 16384
D = 64
H = 32
G = 18
GP = 20
NG = 5
KG = GP // NG
BT = 1024


def _fused_kernel(x_ref, gv_ref, wbig_ref, b1_ref, w2s_ref, b2_ref,
                  w3w_ref, b3w_ref, wa_ref, ba_ref, e_ref, bg_ref, out_ref):
    x = x_ref[:]
    gv = gv_ref[:]

    logits = jnp.dot(x, wa_ref[:], preferred_element_type=jnp.float32) + ba_ref[:]
    m = jnp.max(logits, axis=-1, keepdims=True)
    ex = jnp.exp(logits - m)
    gw = ex / jnp.sum(ex, axis=-1, keepdims=True)
    c = gw * gv
    cexp = jnp.dot(c, e_ref[:], preferred_element_type=jnp.float32)

    p = jnp.dot(x, wbig_ref[:], preferred_element_type=jnp.float32)

    acc = (p[:, GP * 2 * H:]
           + jnp.dot(c, b3w_ref[:], preferred_element_type=jnp.float32)
           + bg_ref[:])
    for t in range(NG):
        s1 = t * KG * 2 * H
        s2 = t * KG * H
        h1 = jnp.maximum(p[:, s1:s1 + KG * 2 * H] + b1_ref[:, s1:s1 + KG * 2 * H], 0.0)
        h2 = jnp.maximum(
            jnp.dot(h1, w2s_ref[t], preferred_element_type=jnp.float32)
            + b2_ref[:, s2:s2 + KG * H], 0.0)
        u = h2 * cexp[:, s2:s2 + KG * H]
        acc += jnp.dot(u, w3w_ref[s2:s2 + KG * H, :],
                       preferred_element_type=jnp.float32)
    out_ref[:] = jnp.maximum(acc, 0.0)


def kernel(item_features, genre_vectors, W1, b1, W2, b2, W3, b3, Wa, ba, Wg, bg):
    pad_g = GP - G
    w1r = jnp.transpose(W1, (1, 0, 2)).reshape(D, G * 2 * H)
    w1p = jnp.pad(w1r, ((0, 0), (0, pad_g * 2 * H)))
    wbig = jnp.concatenate([w1p, Wg[:D]], axis=1)
    b1p = jnp.pad(b1.reshape(1, G * 2 * H), ((0, 0), (0, pad_g * 2 * H)))

    w2p = jnp.pad(W2, ((0, pad_g), (0, 0), (0, 0)))
    eyek = jnp.eye(KG, dtype=jnp.float32)
    w2grp = w2p.reshape(NG, KG, 2 * H, H)
    w2s = (w2grp[:, :, :, None, :] * eyek[None, :, None, :, None]
           ).reshape(NG, KG * 2 * H, KG * H)
    b2p = jnp.pad(b2.reshape(1, G * H), ((0, 0), (0, pad_g * H)))

    w3w = jnp.pad(W3.reshape(G * H, H) @ Wg[D:], ((0, pad_g * H), (0, 0)))
    b3w = b3.reshape(G, H) @ Wg[D:]
    e_expand = jnp.pad(jnp.repeat(jnp.eye(G, dtype=jnp.float32), H, axis=1),
                       ((0, 0), (0, pad_g * H)))
    bar = ba.reshape(1, G)
    bgr = bg.reshape(1, D)

    grid = (B // BT,)
    full = lambda i: (0, 0)
    out = pl.pallas_call(
        _fused_kernel,
        grid=grid,
        in_specs=[
            pl.BlockSpec((BT, D), lambda i: (i, 0)),
            pl.BlockSpec((BT, G), lambda i: (i, 0)),
            pl.BlockSpec((D, GP * 2 * H + D), full),
            pl.BlockSpec((1, GP * 2 * H), full),
            pl.BlockSpec((NG, KG * 2 * H, KG * H), lambda i: (0, 0, 0)),
            pl.BlockSpec((1, GP * H), full),
            pl.BlockSpec((GP * H, D), full),
            pl.BlockSpec((G, D), full),
            pl.BlockSpec((D, G), full),
            pl.BlockSpec((1, G), full),
            pl.BlockSpec((G, GP * H), full),
            pl.BlockSpec((1, D), full),
        ],
        out_specs=pl.BlockSpec((BT, D), lambda i: (i, 0)),
        out_shape=jax.ShapeDtypeStruct((B, D), jnp.float32),
        compiler_params=pltpu.CompilerParams(
            dimension_semantics=("arbitrary",),
        ),
    )(item_features, genre_vectors, wbig, b1p, w2s, b2p, w3w, b3w,
      Wa, bar, e_expand, bgr)
    return out


# speedup vs baseline: 1.3934x; 1.0589x over previous
import jax
import jax.numpy as jnp
from jax.experimental import pallas as pl
from jax.experimental.pallas import tpu as pltpu

B = 16384
D = 64
H = 32
G = 18
GP = 20
NG = 5
KG = GP // NG
BT = 2048


def _fused_kernel(x_ref, gv_ref, wbig_ref, b1_ref, w2s_ref, b2_ref,
                  w3w_ref, b3w_ref, wa_ref, ba_ref, e_ref, bg_ref, out_ref):
    x = x_ref[:]
    gv = gv_ref[:]

    logits = jnp.dot(x, wa_ref[:], preferred_element_type=jnp.float32) + ba_ref[:]
    m = jnp.max(logits, axis=-1, keepdims=True)
    ex = jnp.exp(logits - m)
    gw = ex / jnp.sum(ex, axis=-1, keepdims=True)
    c = gw * gv
    cexp = jnp.dot(c, e_ref[:], preferred_element_type=jnp.float32)

    p = jnp.dot(x, wbig_ref[:], preferred_element_type=jnp.float32)

    acc = (p[:, GP * 2 * H:]
           + jnp.dot(c, b3w_ref[:], preferred_element_type=jnp.float32)
           + bg_ref[:])
    for t in range(NG):
        s1 = t * KG * 2 * H
        s2 = t * KG * H
        h1 = jnp.maximum(p[:, s1:s1 + KG * 2 * H] + b1_ref[:, s1:s1 + KG * 2 * H], 0.0)
        h2 = jnp.maximum(
            jnp.dot(h1, w2s_ref[t], preferred_element_type=jnp.float32)
            + b2_ref[:, s2:s2 + KG * H], 0.0)
        u = h2 * cexp[:, s2:s2 + KG * H]
        acc += jnp.dot(u, w3w_ref[s2:s2 + KG * H, :],
                       preferred_element_type=jnp.float32)
    out_ref[:] = jnp.maximum(acc, 0.0)


def kernel(item_features, genre_vectors, W1, b1, W2, b2, W3, b3, Wa, ba, Wg, bg):
    pad_g = GP - G
    w1r = jnp.transpose(W1, (1, 0, 2)).reshape(D, G * 2 * H)
    w1p = jnp.pad(w1r, ((0, 0), (0, pad_g * 2 * H)))
    wbig = jnp.concatenate([w1p, Wg[:D]], axis=1)
    b1p = jnp.pad(b1.reshape(1, G * 2 * H), ((0, 0), (0, pad_g * 2 * H)))

    w2p = jnp.pad(W2, ((0, pad_g), (0, 0), (0, 0)))
    eyek = jnp.eye(KG, dtype=jnp.float32)
    w2grp = w2p.reshape(NG, KG, 2 * H, H)
    w2s = (w2grp[:, :, :, None, :] * eyek[None, :, None, :, None]
           ).reshape(NG, KG * 2 * H, KG * H)
    b2p = jnp.pad(b2.reshape(1, G * H), ((0, 0), (0, pad_g * H)))

    w3w = jnp.pad(W3.reshape(G * H, H) @ Wg[D:], ((0, pad_g * H), (0, 0)))
    b3w = b3.reshape(G, H) @ Wg[D:]
    e_expand = jnp.pad(jnp.repeat(jnp.eye(G, dtype=jnp.float32), H, axis=1),
                       ((0, 0), (0, pad_g * H)))
    bar = ba.reshape(1, G)
    bgr = bg.reshape(1, D)

    grid = (B // BT,)
    full = lambda i: (0, 0)
    out = pl.pallas_call(
        _fused_kernel,
        grid=grid,
        in_specs=[
            pl.BlockSpec((BT, D), lambda i: (i, 0)),
            pl.BlockSpec((BT, G), lambda i: (i, 0)),
            pl.BlockSpec((D, GP * 2 * H + D), full),
            pl.BlockSpec((1, GP * 2 * H), full),
            pl.BlockSpec((NG, KG * 2 * H, KG * H), lambda i: (0, 0, 0)),
            pl.BlockSpec((1, GP * H), full),
            pl.BlockSpec((GP * H, D), full),
            pl.BlockSpec((G, D), full),
            pl.BlockSpec((D, G), full),
            pl.BlockSpec((1, G), full),
            pl.BlockSpec((G, GP * H), full),
            pl.BlockSpec((1, D), full),
        ],
        out_specs=pl.BlockSpec((BT, D), lambda i: (i, 0)),
        out_shape=jax.ShapeDtypeStruct((B, D), jnp.float32),
        compiler_params=pltpu.CompilerParams(
            dimension_semantics=("arbitrary",),
        ),
    )(item_features, genre_vectors, wbig, b1p, w2s, b2p, w3w, b3w,
      Wa, bar, e_expand, bgr)
    return out
